# baseline (speedup 1.0000x reference)
import functools

import jax
import jax.numpy as jnp
from jax import lax
from jax.experimental import pallas as pl
from jax.experimental.pallas import tpu as pltpu
from jax.experimental.pallas import tpu_sc as plsc

_N = 10000
_E = 320000
_D = 128
_H = 128

_NC = 2
_NS = 16
_NW = _NC * _NS

_NSLAB_G = 10
_SEG = _E // _NSLAB_G
_EPW = _SEG // _NW
_NSLAB_S = 5

_CH = 128
_CHUNKS = ([(j * _CH, _CH) for j in range(_EPW // _CH)]
           + [((_EPW // _CH) * _CH, _EPW % _CH)])
_TAIL = _EPW % _CH

_RPS = 632
_NP = _RPS * _NS

_MESH = dict(core_axis_name="c", subcore_axis_name="s", num_cores=_NC,
             num_subcores=_NS)


def _gather_diff_body(state_h, nstate_h, src_h, dst_h, out_h,
                      idx, rows, sem):
  wid = lax.axis_index("s") * _NC + lax.axis_index("c")
  base0 = wid * _EPW

  pltpu.sync_copy(dst_h.at[pl.ds(base0, _EPW)], idx)
  cps = [
      pltpu.async_copy(nstate_h.at[idx.at[pl.ds(off, k)]],
                       rows.at[pl.ds(off, k)], sem)
      for off, k in _CHUNKS
  ]
  for cp in cps:
    cp.wait()
  pltpu.sync_copy(src_h.at[pl.ds(base0, _EPW)], idx)
  cps = [
      pltpu.async_copy(state_h.at[idx.at[pl.ds(off, k)]],
                       rows.at[pl.ds(off, k)], sem, add=True)
      for off, k in _CHUNKS
  ]
  for cp in cps:
    cp.wait()
  pltpu.sync_copy(rows, out_h.at[pl.ds(base0, _EPW)])


def _make_gather_diff():
  return functools.partial(
      pl.kernel,
      out_type=jax.ShapeDtypeStruct((_SEG, _D), jnp.float32),
      mesh=plsc.VectorSubcoreMesh(**_MESH),
      scratch_types=[
          pltpu.VMEM((_EPW,), jnp.int32),
          pltpu.VMEM((_EPW, _D), jnp.float32),
          pltpu.SemaphoreType.DMA,
      ],
  )(_gather_diff_body)


_SLOTS = 3


def _scatter_body(msg0_h, msg1_h, dst0_h, dst1_h, zeros_h, out_h,
                  i0, i1, i2, it, r0b, r1b, r2b, sem, acc):
  c = lax.axis_index("c")
  s = lax.axis_index("s")
  wid = s * _NC + c
  r0 = s * _RPS
  idxbufs = (i0, i1, i2)
  rowbufs = (r0b, r1b, r2b)
  pltpu.sync_copy(zeros_h.at[pl.ds(r0, _RPS)], acc.at[pl.ds(r0, _RPS)])
  plsc.subcore_barrier()

  base0 = wid * _EPW
  cps = []
  for mh, dh in ((msg0_h, dst0_h), (msg1_h, dst1_h)):
    for off, k in _CHUNKS:
      ci = len(cps)
      slot = ci % _SLOTS
      if ci >= _SLOTS:
        cps[ci - _SLOTS].wait()
      idx = idxbufs[slot] if k == _CH else it
      rows = rowbufs[slot] if k == _CH else rowbufs[slot].at[pl.ds(0, k)]
      pltpu.sync_copy(dh.at[pl.ds(base0 + off, k)], idx)
      pltpu.sync_copy(mh.at[pl.ds(base0 + off, k)], rows)
      cps.append(pltpu.async_copy(rows, acc.at[idx], sem, add=True))
  for ci in range(len(cps) - _SLOTS, len(cps)):
    cps[ci].wait()

  plsc.subcore_barrier()
  pltpu.sync_copy(acc.at[pl.ds(r0, _RPS)], out_h.at[c, pl.ds(r0, _RPS)])


def _make_scatter():
  return functools.partial(
      pl.kernel,
      out_type=jax.ShapeDtypeStruct((_NC, _NP, _D), jnp.float32),
      mesh=plsc.VectorSubcoreMesh(**_MESH),
      scratch_types=[
          pltpu.VMEM((_CH,), jnp.int32),
          pltpu.VMEM((_CH,), jnp.int32),
          pltpu.VMEM((_CH,), jnp.int32),
          pltpu.VMEM((_TAIL,), jnp.int32),
          pltpu.VMEM((_CH, _D), jnp.float32),
          pltpu.VMEM((_CH, _D), jnp.float32),
          pltpu.VMEM((_CH, _D), jnp.float32),
          pltpu.SemaphoreType.DMA,
          pltpu.VMEM_SHARED((_NP, _D), jnp.float32),
      ],
  )(_scatter_body)


_BM = 1280


def _dot(a, b):
  return jnp.dot(a.astype(jnp.bfloat16), b.astype(jnp.bfloat16),
                 preferred_element_type=jnp.float32)


def _mlp_body(diff_ref, ef_ref, mw1d, mw1e, mb1, mw2, mb2,
              aw1d, aw1e, ab1, aw2, ab2, out_ref):
  d = diff_ref[...]
  ef = ef_ref[...]
  h = jnp.maximum(_dot(d, mw1d[...]) + _dot(ef, mw1e[...]) + mb1[...], 0.0)
  m = _dot(h, mw2[...]) + mb2[...]
  ha = jnp.maximum(_dot(d, aw1d[...]) + _dot(ef, aw1e[...]) + ab1[...], 0.0)
  a = jax.nn.sigmoid(_dot(ha, aw2[...]) + ab2[...])
  out_ref[...] = m * a


def _mlp(diff, ef, weights, interpret=False):
  full = pl.BlockSpec((_H, _H), lambda i: (0, 0))
  bias = pl.BlockSpec((1, _H), lambda i: (0, 0))
  blk = pl.BlockSpec((_BM, _H), lambda i: (i, 0))
  n = diff.shape[0]
  return pl.pallas_call(
      _mlp_body,
      grid=(n // _BM,),
      in_specs=[blk, blk, full, full, bias, full, bias,
                full, full, bias, full, bias],
      out_specs=blk,
      out_shape=jax.ShapeDtypeStruct((n, _H), jnp.float32),
      interpret=interpret,
  )(diff, ef, *weights)


_BN = 1000


def _gru_body(*refs):
  p_refs = refs[:_NSLAB_S]
  st_ref, wih, whh, bih, bhh, out_ref = refs[_NSLAB_S:]
  sm = p_refs[0][0] + p_refs[0][1]
  for p in p_refs[1:]:
    sm = sm + p[0] + p[1]
  st = st_ref[...]
  gi = jnp.dot(sm, wih[...]) + bih[...]
  gh = jnp.dot(st, whh[...]) + bhh[...]
  r = jax.nn.sigmoid(gi[:, :_H] + gh[:, :_H])
  z = jax.nn.sigmoid(gi[:, _H:2 * _H] + gh[:, _H:2 * _H])
  n = jnp.tanh(gi[:, 2 * _H:] + r * gh[:, 2 * _H:])
  out_ref[...] = (1.0 - z) * n + z * st


def _gru(partials, state, wihT, whhT, bih, bhh, interpret=False):
  pspec = pl.BlockSpec((_NC, _BN, _H), lambda i: (0, i, 0))
  return pl.pallas_call(
      _gru_body,
      grid=(_N // _BN,),
      in_specs=[pspec] * _NSLAB_S + [
          pl.BlockSpec((_BN, _H), lambda i: (i, 0)),
          pl.BlockSpec((_H, 3 * _H), lambda i: (0, 0)),
          pl.BlockSpec((_H, 3 * _H), lambda i: (0, 0)),
          pl.BlockSpec((1, 3 * _H), lambda i: (0, 0)),
          pl.BlockSpec((1, 3 * _H), lambda i: (0, 0)),
      ],
      out_specs=pl.BlockSpec((_BN, _H), lambda i: (i, 0)),
      out_shape=jax.ShapeDtypeStruct((_N, _H), jnp.float32),
      interpret=interpret,
  )(*partials, state, wihT, whhT, bih, bhh)


def kernel(node_feat, edge, edge_feat, msg_W1, msg_b1, msg_W2, msg_b2,
           att_W1, att_b1, att_W2, att_b2, gru_Wih, gru_Whh, gru_bih,
           gru_bhh):
  src = edge[:, 0]
  dst = edge[:, 1]
  nstate = -node_feat
  zeros = jnp.zeros((_NP, _D), jnp.float32)

  weights = (
      msg_W1[:, :_D].T, msg_W1[:, _D:].T, msg_b1.reshape(1, _H),
      msg_W2.T, msg_b2.reshape(1, _H),
      att_W1[:, :_D].T, att_W1[:, _D:].T, att_b1.reshape(1, _H),
      att_W2.T, att_b2.reshape(1, _H))

  gather = _make_gather_diff()
  scatter = _make_scatter()

  msgs = []
  dsts = []
  for g in range(_NSLAB_G):
    lo = g * _SEG
    src_s = lax.slice_in_dim(src, lo, lo + _SEG)
    dst_s = lax.slice_in_dim(dst, lo, lo + _SEG)
    ef_s = lax.slice_in_dim(edge_feat, lo, lo + _SEG)
    diff = gather(node_feat, nstate, src_s, dst_s)
    msgs.append(_mlp(diff, ef_s, weights))
    dsts.append(dst_s)

  partials = []
  for s in range(_NSLAB_S):
    partials.append(scatter(msgs[2 * s], msgs[2 * s + 1],
                            dsts[2 * s], dsts[2 * s + 1], zeros))

  return _gru(partials, node_feat, gru_Wih.T, gru_Whh.T,
              gru_bih.reshape(1, 3 * _H), gru_bhh.reshape(1, 3 * _H))


# speedup vs baseline: 1.0151x; 1.0151x over previous
import functools

import jax
import jax.numpy as jnp
from jax import lax
from jax.experimental import pallas as pl
from jax.experimental.pallas import tpu as pltpu
from jax.experimental.pallas import tpu_sc as plsc

_N = 10000
_E = 320000
_D = 128
_H = 128

_NC = 2
_NS = 16
_NW = _NC * _NS

_NSLAB = 5
_SE = _E // _NSLAB
_EPW = _SE // _NW
_HEPW = _EPW // 2

_CH = 128
_CHUNKS_H = ([(j * _CH, _CH) for j in range(_HEPW // _CH)]
             + [((_HEPW // _CH) * _CH, _HEPW % _CH)])

_RPS = 632
_NP = _RPS * _NS

_MESH = dict(core_axis_name="c", subcore_axis_name="s", num_cores=_NC,
             num_subcores=_NS)


def _gather_diff_body(state_h, nstate_h, src_h, dst_h, out_h,
                      idx, rows, sem):
  wid = lax.axis_index("s") * _NC + lax.axis_index("c")
  base0 = wid * _EPW

  for h in range(2):
    hb = base0 + h * _HEPW
    pltpu.sync_copy(dst_h.at[pl.ds(hb, _HEPW)], idx)
    cps = [
        pltpu.async_copy(nstate_h.at[idx.at[pl.ds(off, k)]],
                         rows.at[pl.ds(off, k)], sem)
        for off, k in _CHUNKS_H
    ]
    for cp in cps:
      cp.wait()
    pltpu.sync_copy(src_h.at[pl.ds(hb, _HEPW)], idx)
    cps = [
        pltpu.async_copy(state_h.at[idx.at[pl.ds(off, k)]],
                         rows.at[pl.ds(off, k)], sem, add=True)
        for off, k in _CHUNKS_H
    ]
    for cp in cps:
      cp.wait()
    pltpu.sync_copy(rows, out_h.at[pl.ds(hb, _HEPW)])


def _make_gather_diff():
  return functools.partial(
      pl.kernel,
      out_type=jax.ShapeDtypeStruct((_SE, _D), jnp.float32),
      mesh=plsc.VectorSubcoreMesh(**_MESH),
      scratch_types=[
          pltpu.VMEM((_HEPW,), jnp.int32),
          pltpu.VMEM((_HEPW, _D), jnp.float32),
          pltpu.SemaphoreType.DMA,
      ],
  )(_gather_diff_body)


_SLOTS = 3
_SCHUNKS = ([(j * _CH, _CH) for j in range(_EPW // _CH)]
            + [((_EPW // _CH) * _CH, _EPW % _CH)])
_STAIL = _EPW % _CH


def _scatter_body(msg_h, dst_h, zeros_h, out_h,
                  i0, i1, i2, it, r0b, r1b, r2b, sem, acc):
  c = lax.axis_index("c")
  s = lax.axis_index("s")
  wid = s * _NC + c
  r0 = s * _RPS
  idxbufs = (i0, i1, i2)
  rowbufs = (r0b, r1b, r2b)
  pltpu.sync_copy(zeros_h.at[pl.ds(r0, _RPS)], acc.at[pl.ds(r0, _RPS)])
  plsc.subcore_barrier()

  base0 = wid * _EPW
  cps = []
  for off, k in _SCHUNKS:
    ci = len(cps)
    slot = ci % _SLOTS
    if ci >= _SLOTS:
      cps[ci - _SLOTS].wait()
    idx = idxbufs[slot] if k == _CH else it
    rows = rowbufs[slot] if k == _CH else rowbufs[slot].at[pl.ds(0, k)]
    pltpu.sync_copy(dst_h.at[pl.ds(base0 + off, k)], idx)
    pltpu.sync_copy(msg_h.at[pl.ds(base0 + off, k)], rows)
    cps.append(pltpu.async_copy(rows, acc.at[idx], sem, add=True))
  for ci in range(len(cps) - _SLOTS, len(cps)):
    cps[ci].wait()

  plsc.subcore_barrier()
  pltpu.sync_copy(acc.at[pl.ds(r0, _RPS)], out_h.at[c, pl.ds(r0, _RPS)])


def _make_scatter():
  return functools.partial(
      pl.kernel,
      out_type=jax.ShapeDtypeStruct((_NC, _NP, _D), jnp.float32),
      mesh=plsc.VectorSubcoreMesh(**_MESH),
      scratch_types=[
          pltpu.VMEM((_CH,), jnp.int32),
          pltpu.VMEM((_CH,), jnp.int32),
          pltpu.VMEM((_CH,), jnp.int32),
          pltpu.VMEM((_STAIL,), jnp.int32),
          pltpu.VMEM((_CH, _D), jnp.float32),
          pltpu.VMEM((_CH, _D), jnp.float32),
          pltpu.VMEM((_CH, _D), jnp.float32),
          pltpu.SemaphoreType.DMA,
          pltpu.VMEM_SHARED((_NP, _D), jnp.float32),
      ],
  )(_scatter_body)


_BM = 1280


def _dot(a, b):
  return jnp.dot(a.astype(jnp.bfloat16), b.astype(jnp.bfloat16),
                 preferred_element_type=jnp.float32)


def _mlp_body(diff_ref, ef_ref, mw1d, mw1e, mb1, mw2, mb2,
              aw1d, aw1e, ab1, aw2, ab2, out_ref):
  d = diff_ref[...]
  ef = ef_ref[...]
  h = jnp.maximum(_dot(d, mw1d[...]) + _dot(ef, mw1e[...]) + mb1[...], 0.0)
  m = _dot(h, mw2[...]) + mb2[...]
  ha = jnp.maximum(_dot(d, aw1d[...]) + _dot(ef, aw1e[...]) + ab1[...], 0.0)
  a = jax.nn.sigmoid(_dot(ha, aw2[...]) + ab2[...])
  out_ref[...] = m * a


def _mlp(diff, ef, weights, interpret=False):
  full = pl.BlockSpec((_H, _H), lambda i: (0, 0))
  bias = pl.BlockSpec((1, _H), lambda i: (0, 0))
  blk = pl.BlockSpec((_BM, _H), lambda i: (i, 0))
  n = diff.shape[0]
  return pl.pallas_call(
      _mlp_body,
      grid=(n // _BM,),
      in_specs=[blk, blk, full, full, bias, full, bias,
                full, full, bias, full, bias],
      out_specs=blk,
      out_shape=jax.ShapeDtypeStruct((n, _H), jnp.float32),
      interpret=interpret,
  )(diff, ef, *weights)


_BN = 1000


def _gru_body(*refs):
  p_refs = refs[:_NSLAB]
  st_ref, wih, whh, bih, bhh, out_ref = refs[_NSLAB:]
  sm = p_refs[0][0] + p_refs[0][1]
  for p in p_refs[1:]:
    sm = sm + p[0] + p[1]
  st = st_ref[...]
  gi = jnp.dot(sm, wih[...]) + bih[...]
  gh = jnp.dot(st, whh[...]) + bhh[...]
  r = jax.nn.sigmoid(gi[:, :_H] + gh[:, :_H])
  z = jax.nn.sigmoid(gi[:, _H:2 * _H] + gh[:, _H:2 * _H])
  n = jnp.tanh(gi[:, 2 * _H:] + r * gh[:, 2 * _H:])
  out_ref[...] = (1.0 - z) * n + z * st


def _gru(partials, state, wihT, whhT, bih, bhh, interpret=False):
  pspec = pl.BlockSpec((_NC, _BN, _H), lambda i: (0, i, 0))
  return pl.pallas_call(
      _gru_body,
      grid=(_N // _BN,),
      in_specs=[pspec] * _NSLAB + [
          pl.BlockSpec((_BN, _H), lambda i: (i, 0)),
          pl.BlockSpec((_H, 3 * _H), lambda i: (0, 0)),
          pl.BlockSpec((_H, 3 * _H), lambda i: (0, 0)),
          pl.BlockSpec((1, 3 * _H), lambda i: (0, 0)),
          pl.BlockSpec((1, 3 * _H), lambda i: (0, 0)),
      ],
      out_specs=pl.BlockSpec((_BN, _H), lambda i: (i, 0)),
      out_shape=jax.ShapeDtypeStruct((_N, _H), jnp.float32),
      interpret=interpret,
  )(*partials, state, wihT, whhT, bih, bhh)


def kernel(node_feat, edge, edge_feat, msg_W1, msg_b1, msg_W2, msg_b2,
           att_W1, att_b1, att_W2, att_b2, gru_Wih, gru_Whh, gru_bih,
           gru_bhh):
  src = edge[:, 0]
  dst = edge[:, 1]
  nstate = -node_feat
  zeros = jnp.zeros((_NP, _D), jnp.float32)

  weights = (
      msg_W1[:, :_D].T, msg_W1[:, _D:].T, msg_b1.reshape(1, _H),
      msg_W2.T, msg_b2.reshape(1, _H),
      att_W1[:, :_D].T, att_W1[:, _D:].T, att_b1.reshape(1, _H),
      att_W2.T, att_b2.reshape(1, _H))

  gather = _make_gather_diff()
  scatter = _make_scatter()

  partials = []
  for g in range(_NSLAB):
    lo = g * _SE
    src_s = lax.slice_in_dim(src, lo, lo + _SE)
    dst_s = lax.slice_in_dim(dst, lo, lo + _SE)
    ef_s = lax.slice_in_dim(edge_feat, lo, lo + _SE)
    diff = gather(node_feat, nstate, src_s, dst_s)
    msg = _mlp(diff, ef_s, weights)
    partials.append(scatter(msg, dst_s, zeros))

  return _gru(partials, node_feat, gru_Wih.T, gru_Whh.T,
              gru_bih.reshape(1, 3 * _H), gru_bhh.reshape(1, 3 * _H))
